# Initial kernel scaffold; baseline (speedup 1.0000x reference)
import jax
import jax.numpy as jnp
from jax.experimental import pallas as pl


def kernel(batch, emb_table, W, b):
    raise NotImplementedError("write your pallas kernel here")


# speedup vs baseline: 6.0263x; 6.0263x over previous
import functools

import jax
import jax.numpy as jnp
from jax import lax
from jax.experimental import pallas as pl
from jax.experimental.pallas import tpu as pltpu
from jax.experimental.pallas import tpu_sc as plsc

NC, NS = 2, 16
NW = NC * NS
OUT_D = 10
PAD_D = 16

BATCH, SEQ = 4096, 200
TOTAL = BATCH * SEQ
N_PER_W = TOTAL // NW
G = 128
K = 20
CHUNK = G * K
N_CHUNKS = N_PER_W // CHUNK


def _fuse_table_body(emb_ref, w_ref, b_ref, out_ref):
    out_ref[...] = (
        jnp.dot(emb_ref[...], w_ref[...], preferred_element_type=jnp.float32)
        + b_ref[...]
    )


def _pad_wb(W, b):
    wp = jnp.zeros((W.shape[0], PAD_D), jnp.float32).at[:, :OUT_D].set(W)
    bp = jnp.zeros((1, PAD_D), jnp.float32).at[0, :OUT_D].set(b)
    return wp, bp


def _gather_body(fused_hbm, idx_hbm, out_hbm, idx_v, rows_v, sem):
    wid = lax.axis_index("s") * NC + lax.axis_index("c")
    base = wid * N_PER_W

    def chunk(g, carry):
        off = base + g * CHUNK
        pltpu.sync_copy(idx_hbm.at[pl.ds(off // G, K)], idx_v)
        copies = [
            pltpu.async_copy(
                fused_hbm.at[idx_v.at[j]], rows_v.at[pl.ds(j * G, G)], sem
            )
            for j in range(K)
        ]
        for c in copies:
            c.wait()
        pltpu.sync_copy(rows_v, out_hbm.at[pl.ds(off, CHUNK)])
        return carry

    lax.fori_loop(0, N_CHUNKS, chunk, 0)


def kernel(batch, emb_table, W, b):
    wp, bp = _pad_wb(W, b)
    fused = pl.pallas_call(
        _fuse_table_body,
        out_shape=jax.ShapeDtypeStruct((emb_table.shape[0], PAD_D), jnp.float32),
    )(emb_table, wp, bp)

    idx = batch.reshape(TOTAL // G, G)

    mesh = plsc.VectorSubcoreMesh(core_axis_name="c", subcore_axis_name="s")
    gathered = pl.kernel(
        _gather_body,
        out_type=jax.ShapeDtypeStruct((TOTAL, PAD_D), jnp.float32),
        mesh=mesh,
        scratch_types=[
            pltpu.VMEM((K, G), jnp.int32),
            pltpu.VMEM((CHUNK, PAD_D), jnp.float32),
            pltpu.SemaphoreType.DMA,
        ],
        compiler_params=pltpu.CompilerParams(use_tc_tiling_on_sc=False),
    )(fused, idx)

    return gathered[:, :OUT_D].reshape(BATCH, SEQ, OUT_D)
